# Initial kernel scaffold; baseline (speedup 1.0000x reference)
import jax
import jax.numpy as jnp
from jax.experimental import pallas as pl


def kernel(queries, keys):
    raise NotImplementedError("write your pallas kernel here")


# speedup vs baseline: 10.9595x; 10.9595x over previous
import jax
import jax.numpy as jnp
from jax.experimental import pallas as pl

QN, DN, TK = 16, 64, 8
NEG = float("-inf")


def _stage1_body(q_ref, k_ref, cv_ref, ci_ref, mns_ref, *, bk, kn):
    i = pl.program_id(0)
    q = q_ref[...]
    k = k_ref[...]
    s = jax.lax.dot_general(q, k, (((1,), (1,)), ((), ())),
                            preferred_element_type=jnp.float32)
    ks = jnp.sum(k * k, axis=1)
    bmax = jnp.max(ks)

    bmax2 = jnp.reshape(bmax, (1, 1))

    @pl.when(i == 0)
    def _():
        mns_ref[...] = bmax2

    @pl.when(i != 0)
    def _():
        mns_ref[...] = jnp.maximum(mns_ref[...], bmax2)

    iota = jax.lax.broadcasted_iota(jnp.int32, (QN, bk), 1)
    base = i * bk
    vals, idxs = [], []
    for _ in range(TK + 1):
        rv = jnp.max(s, axis=1, keepdims=True)
        p = jnp.min(jnp.where(s == rv, iota, kn), axis=1, keepdims=True)
        vals.append(rv)
        idxs.append(p + base)
        s = jnp.where(iota == p, NEG, s)
    pad_v = [jnp.full((QN, 1), NEG, jnp.float32)] * (16 - (TK + 1))
    pad_i = [jnp.zeros((QN, 1), jnp.int32)] * (16 - (TK + 1))
    cv_ref[0] = jnp.concatenate(vals + pad_v, axis=1)
    ci_ref[0] = jnp.concatenate(idxs + pad_i, axis=1)


def _stage2_body(cv_ref, ci_ref, q_ref, mns_ref, sc_ref, dip_ref, it_ref, *, m2):
    c_vals = cv_ref[...]
    c_idx = ci_ref[...]
    iota = jax.lax.broadcasted_iota(jnp.int32, (QN, m2), 1)
    vals, gidx = [], []
    for _ in range(TK + 1):
        rv = jnp.max(c_vals, axis=1, keepdims=True)
        p = jnp.min(jnp.where(c_vals == rv, iota, m2), axis=1, keepdims=True)
        sel = iota == p
        g = jnp.sum(jnp.where(sel, c_idx, 0), axis=1, keepdims=True)
        vals.append(rv)
        gidx.append(g)
        c_vals = jnp.where(sel, NEG, c_vals)
    v = jnp.concatenate(vals, axis=1)
    g = jnp.concatenate(gidx, axis=1)
    q = q_ref[...]
    mns = mns_ref[...][0, 0]
    qsq = jnp.sum(q * q, axis=1, keepdims=True)
    c = qsq + mns
    dl2 = c - 2.0 * v
    dip_ref[...] = (c - dl2) * 0.5 / mns
    sc_ref[...] = v[:, :TK] / mns
    it_ref[...] = g[:, :TK]


def kernel(queries, keys):
    kn = keys.shape[0]
    bk = 2000
    assert kn % bk == 0
    nb = kn // bk
    m2 = nb * 16

    import functools
    cv, ci, mns = pl.pallas_call(
        functools.partial(_stage1_body, bk=bk, kn=kn),
        grid=(nb,),
        in_specs=[
            pl.BlockSpec((QN, DN), lambda i: (0, 0)),
            pl.BlockSpec((bk, DN), lambda i: (i, 0)),
        ],
        out_specs=[
            pl.BlockSpec((1, QN, 16), lambda i: (i, 0, 0)),
            pl.BlockSpec((1, QN, 16), lambda i: (i, 0, 0)),
            pl.BlockSpec((1, 1), lambda i: (0, 0)),
        ],
        out_shape=[
            jax.ShapeDtypeStruct((nb, QN, 16), jnp.float32),
            jax.ShapeDtypeStruct((nb, QN, 16), jnp.int32),
            jax.ShapeDtypeStruct((1, 1), jnp.float32),
        ],
    )(queries, keys)

    cvt = jnp.transpose(cv, (1, 0, 2)).reshape(QN, m2)
    cit = jnp.transpose(ci, (1, 0, 2)).reshape(QN, m2)

    sc, dip, it = pl.pallas_call(
        functools.partial(_stage2_body, m2=m2),
        out_shape=[
            jax.ShapeDtypeStruct((QN, TK), jnp.float32),
            jax.ShapeDtypeStruct((QN, TK + 1), jnp.float32),
            jax.ShapeDtypeStruct((QN, TK), jnp.int32),
        ],
    )(cvt, cit, queries, mns)
    return sc, dip, it


# speedup vs baseline: 36.4003x; 3.3213x over previous
import jax
import jax.numpy as jnp
from jax import lax
from jax.experimental import pallas as pl
from jax.experimental.pallas import tpu as pltpu
from jax.experimental.pallas import tpu_sc as plsc

QN, DN, TK = 16, 64, 8
KN = 1_000_000
BKP = 4096
NB1 = 245
SP = NB1 * BKP
NEG = float("-inf")

NR = 16
RNG = SP // NR
CHP = RNG // 10
NPIECE = 10
GRP = 14
NGRP = CHP // (16 * GRP)


def _stage1_body(q_ref, kt_ref, s_ref, aux_ref):
    i = pl.program_id(0)
    q = q_ref[...]
    kt = kt_ref[...]
    s = lax.dot_general(q, kt, (((1,), (0,)), ((), ())),
                        preferred_element_type=jnp.float32)
    col = lax.broadcasted_iota(jnp.int32, (QN, BKP), 1) + i * BKP
    s_ref[...] = jnp.where(col < KN, s, NEG)

    ks = jnp.sum(kt * kt, axis=0, keepdims=True)
    cid2 = lax.broadcasted_iota(jnp.int32, (1, BKP), 1) + i * BKP
    ks = jnp.where(cid2 < KN, ks, -1.0)
    bcol = jnp.broadcast_to(jnp.reshape(jnp.max(ks), (1, 1)), (QN, 1))

    @pl.when(i == 0)
    def _():
        qsq = jnp.sum(q * q, axis=1, keepdims=True)
        aux_ref[...] = jnp.concatenate(
            [qsq, bcol] + [jnp.zeros((QN, 1), jnp.float32)] * 14, axis=1)

    @pl.when(i != 0)
    def _():
        aux_ref[:, 1:2] = jnp.maximum(aux_ref[:, 1:2], bcol)

    @pl.when(i == NB1 - 1)
    def _():
        aux_ref[:, 0:1] = aux_ref[:, 0:1] + aux_ref[:, 1:2]


def _stage1(queries, keyst):
    return pl.pallas_call(
        _stage1_body,
        grid=(NB1,),
        in_specs=[
            pl.BlockSpec((QN, DN), lambda i: (0, 0)),
            pl.BlockSpec((DN, BKP), lambda i: (0, i)),
        ],
        out_specs=[
            pl.BlockSpec((QN, BKP), lambda i: (0, i)),
            pl.BlockSpec((QN, 16), lambda i: (0, 0)),
        ],
        out_shape=[
            jax.ShapeDtypeStruct((QN, SP), jnp.float32),
            jax.ShapeDtypeStruct((QN, 16), jnp.float32),
        ],
    )(queries, keyst)


def _merge_row(runv, runi, qr, newv, newi):
    sv, si = plsc.sort_key_val(newv, newi, descending=True)
    nrv = lax.rev(sv, (0,))
    nri = lax.rev(si, (0,))
    rv = runv[qr, :]
    ri = runi[qr, :]
    m = rv >= nrv
    cv = jnp.where(m, rv, nrv)
    ci = jnp.where(m, ri, nri)
    mv, mi = plsc.sort_key_val(cv, ci, descending=True)
    runv[qr, :] = mv
    runi[qr, :] = mi


def _thresh(runv, qr):
    full = jnp.full((16,), qr, jnp.int32)
    return plsc.load_gather(runv, [full, jnp.full((16,), 8, jnp.int32)])


def _stage2_body(s_hbm, cv_hbm, ci_hbm, buf, runv, runi):
    g = lax.axis_index("c")
    r = lax.axis_index("s")
    lanes = lax.iota(jnp.int32, 16)
    for qr in range(8):
        runv[qr, :] = jnp.full((16,), NEG, jnp.float32)
        runi[qr, :] = jnp.zeros((16,), jnp.int32)
    lo = r * RNG

    def piece(p, _):
        pltpu.sync_copy(
            s_hbm.at[pl.ds(g * 8, 8), pl.ds(lo + p * CHP, CHP)], buf)

        def group(gi, _):
            base = gi * (16 * GRP)
            for qr in range(8):
                th = _thresh(runv, qr)
                hit = None
                for c in range(GRP):
                    v = buf[qr, pl.ds(base + c * 16, 16)]
                    m = v > th
                    hit = m if hit is None else jnp.logical_or(hit, m)

                @pl.when(jnp.any(hit))
                def _(qr=qr, base=base, p=p):
                    def rescan(c2, _):
                        v2 = buf[qr, pl.ds(base + c2 * 16, 16)]
                        th2 = _thresh(runv, qr)

                        @pl.when(jnp.any(v2 > th2))
                        def _():
                            gidx = lo + p * CHP + base + c2 * 16 + lanes
                            _merge_row(runv, runi, qr, v2, gidx)

                        return 0

                    lax.fori_loop(0, GRP, rescan, 0)

            return 0

        lax.fori_loop(0, NGRP, group, 0)
        return 0

    lax.fori_loop(0, NPIECE, piece, 0)

    for qr in range(8):
        q = g * 8 + qr
        pltpu.sync_copy(runv.at[qr], cv_hbm.at[q, pl.ds(r * 16, 16)])
        pltpu.sync_copy(runi.at[qr], ci_hbm.at[q, pl.ds(r * 16, 16)])


def _stage3_body(cv_hbm, ci_hbm, aux_hbm, sc_hbm, dip_hbm, it_hbm,
                 bufv, bufi, auxv, runv, runi, rowf, rowd, rowi):
    cid = lax.axis_index("c")
    sid = lax.axis_index("s")

    @pl.when(sid < 8)
    def _():
        q = sid * 2 + cid
        pltpu.sync_copy(cv_hbm, bufv)
        pltpu.sync_copy(ci_hbm, bufi)
        pltpu.sync_copy(aux_hbm, auxv)
        runv[0, :] = jnp.full((16,), NEG, jnp.float32)
        runi[0, :] = jnp.zeros((16,), jnp.int32)
        lanes = lax.iota(jnp.int32, 16)
        qv = jnp.full((16,), q, jnp.int32)
        for rr in range(NR):
            rrv = jnp.full((16,), rr * 16, jnp.int32) + lanes
            v = plsc.load_gather(bufv, [qv, rrv])
            ii = plsc.load_gather(bufi, [qv, rrv])
            _merge_row(runv, runi, 0, v, ii)

        c_spl = plsc.load_gather(auxv, [qv, jnp.zeros((16,), jnp.int32)])
        mns_spl = plsc.load_gather(auxv, [qv, jnp.ones((16,), jnp.int32)])
        rv = runv[0, :]
        dl2 = c_spl - 2.0 * rv
        rowd[...] = (c_spl - dl2) * 0.5 / mns_spl
        rowf[...] = rv / mns_spl
        rowi[...] = runi[0, :]
        pltpu.sync_copy(rowf, sc_hbm.at[q])
        pltpu.sync_copy(rowd, dip_hbm.at[q])
        pltpu.sync_copy(rowi, it_hbm.at[q])


def _stage2(scores):
    mesh = plsc.VectorSubcoreMesh(core_axis_name="c", subcore_axis_name="s")
    f = pl.kernel(
        _stage2_body,
        out_type=[
            jax.ShapeDtypeStruct((QN, NR * 16), jnp.float32),
            jax.ShapeDtypeStruct((QN, NR * 16), jnp.int32),
        ],
        mesh=mesh,
        compiler_params=pltpu.CompilerParams(needs_layout_passes=False),
        scratch_types=[
            pltpu.VMEM((8, CHP), jnp.float32),
            pltpu.VMEM((8, 16), jnp.float32),
            pltpu.VMEM((8, 16), jnp.int32),
        ],
    )
    return f(scores)


def _stage3(cv, ci, aux):
    mesh = plsc.VectorSubcoreMesh(core_axis_name="c", subcore_axis_name="s")
    f = pl.kernel(
        _stage3_body,
        out_type=[
            jax.ShapeDtypeStruct((QN, 16), jnp.float32),
            jax.ShapeDtypeStruct((QN, 16), jnp.float32),
            jax.ShapeDtypeStruct((QN, 16), jnp.int32),
        ],
        mesh=mesh,
        compiler_params=pltpu.CompilerParams(needs_layout_passes=False),
        scratch_types=[
            pltpu.VMEM((QN, NR * 16), jnp.float32),
            pltpu.VMEM((QN, NR * 16), jnp.int32),
            pltpu.VMEM((QN, 16), jnp.float32),
            pltpu.VMEM((1, 16), jnp.float32),
            pltpu.VMEM((1, 16), jnp.int32),
            pltpu.VMEM((16,), jnp.float32),
            pltpu.VMEM((16,), jnp.float32),
            pltpu.VMEM((16,), jnp.int32),
        ],
    )
    return f(cv, ci, aux)


def kernel(queries, keys):
    scores, aux = _stage1(queries, keys.T)
    cv, ci = _stage2(scores)
    sc, dip, it = _stage3(cv, ci, aux)
    return sc[:, :TK], dip[:, :TK + 1], it[:, :TK]
